# baseline (speedup 1.0000x reference)
import functools

import jax
import jax.numpy as jnp
from jax import lax
from jax.experimental import pallas as pl
from jax.experimental.pallas import tpu as pltpu
from jax.experimental.pallas import tpu_sc as plsc

_N_QUANTILES = 1000
_NUM_TOKENS = 3
_N = 16384
_L = 200
_D = 64

_NC = 2
_NS = 16
_NW = _NC * _NS
_LANES = 16

_LT = _L // 8
_NT = _N // 128
_NTW = _NT // _NW
_BLK = _NTW * 8 * 128
_TROWS = 1024


def _embed_body(v5, i5, tabt_hbm, out5,
                vblk, iblk, outb, tab_v, lsem, osem0, osem1):
    cid = lax.axis_index("c")
    sid = lax.axis_index("s")
    wid = sid * _NC + cid
    coloff = wid * _NTW * 1024

    pltpu.sync_copy(tabt_hbm, tab_v)

    osem = (osem0, osem1)

    def fire_in(lt, boff):
        pltpu.async_copy(v5.at[lt, pl.ds(coloff, _BLK)],
                         vblk.at[pl.ds(boff, _BLK)], lsem)
        pltpu.async_copy(i5.at[lt, pl.ds(coloff, _BLK)],
                         iblk.at[pl.ds(boff, _BLK)], lsem)

    fire_in(0, 0)

    def lt_body(lt, carry):
        qoff = (lt % 2) * _BLK
        pltpu.make_async_copy(v5.at[lt, pl.ds(coloff, _BLK)],
                              vblk.at[pl.ds(qoff, _BLK)], lsem).wait()
        pltpu.make_async_copy(i5.at[lt, pl.ds(coloff, _BLK)],
                              iblk.at[pl.ds(qoff, _BLK)], lsem).wait()

        @pl.when(lt + 1 < _LT)
        def _():
            fire_in(lt + 1, _BLK - qoff)

        @plsc.parallel_loop(0, _BLK // _LANES, unroll=4)
        def _(m):
            v = vblk[pl.ds(qoff + m * _LANES, _LANES)]
            ind = iblk[pl.ds(qoff + m * _LANES, _LANES)]
            v = jnp.minimum(jnp.maximum(v, 0.0), 1.0)
            q = (v * float(_N_QUANTILES)).astype(jnp.int32) + _NUM_TOKENS
            lk = jnp.where(ind == 0, q, ind)
            lk = jnp.minimum(jnp.maximum(lk, 0),
                             _N_QUANTILES + _NUM_TOKENS - 1)
            iblk[pl.ds(qoff + m * _LANES, _LANES)] = lk

        def ls_body(ls, c3):
            l = lt * 8 + ls

            def dp2_body(dp2, c4):
                for e in range(2):
                    dp = 2 * dp2 + e
                    cnt = (lt * 8 + ls) * 4 + dp

                    @pl.when(cnt >= 2)
                    def _():
                        pltpu.make_async_copy(
                            outb.at[e],
                            out5.at[l, pl.ds(2 * dp, 2),
                                    pl.ds(wid * _NTW, _NTW)],
                            osem[e]).wait()

                    @plsc.parallel_loop(0, _NTW * 8, unroll=8)
                    def _(m):
                        nt4 = m // 8
                        g = m % 8
                        col = iblk[pl.ds(qoff + nt4 * 1024 + ls * 128
                                         + g * _LANES, _LANES)]
                        for dd in range(2):
                            dt = 2 * dp + dd
                            for ds in range(8):
                                x = plsc.load_gather(
                                    tab_v, [col + (dt * 8 + ds) * _TROWS])
                                outb[e, dd, nt4, ds,
                                     pl.ds(g * _LANES, _LANES)] = x
                    pltpu.async_copy(outb.at[e],
                                     out5.at[l, pl.ds(2 * dp, 2),
                                             pl.ds(wid * _NTW, _NTW)],
                                     osem[e])
                return c4

            lax.fori_loop(0, 2, dp2_body, 0)
            return c3

        lax.fori_loop(0, 8, ls_body, 0)
        return carry

    lax.fori_loop(0, _LT, lt_body, 0)

    for e in range(2):
        pltpu.make_async_copy(outb.at[e],
                              out5.at[_L - 1, pl.ds(4 + 2 * e, 2),
                                      pl.ds(wid * _NTW, _NTW)],
                              osem[e]).wait()


@jax.jit
def kernel(values, indicators, table):
    n, l = values.shape
    v5 = (values.reshape(_NT, 128, _LT, 8).transpose(2, 0, 3, 1)
          .reshape(_LT, _NT * 8 * 128))
    i5 = (indicators.reshape(_NT, 128, _LT, 8).transpose(2, 0, 3, 1)
          .reshape(_LT, _NT * 8 * 128))
    tabt = jnp.pad(table, ((0, _TROWS - table.shape[0]), (0, 0))).T.reshape(-1)

    run = functools.partial(
        pl.kernel,
        mesh=plsc.VectorSubcoreMesh(core_axis_name="c", subcore_axis_name="s"),
        compiler_params=pltpu.CompilerParams(use_tc_tiling_on_sc=False,
                                             needs_layout_passes=False),
        out_type=jax.ShapeDtypeStruct((_L, _D // 8, _NT, 8, 128),
                                      jnp.float32),
        scratch_types=[
            pltpu.VMEM((2 * _BLK,), jnp.float32),
            pltpu.VMEM((2 * _BLK,), jnp.int32),
            pltpu.VMEM((2, 2, _NTW, 8, 128), jnp.float32),
            pltpu.VMEM((_D * _TROWS,), jnp.float32),
            pltpu.SemaphoreType.DMA,
            pltpu.SemaphoreType.DMA,
            pltpu.SemaphoreType.DMA,
        ],
    )(_embed_body)

    out5 = run(v5, i5, tabt)
    out = out5.transpose(2, 4, 0, 1, 3).reshape(n, l, _D)
    return out


# speedup vs baseline: 1.0438x; 1.0438x over previous
import functools

import jax
import jax.numpy as jnp
from jax import lax
from jax.experimental import pallas as pl
from jax.experimental.pallas import tpu as pltpu
from jax.experimental.pallas import tpu_sc as plsc

_N_QUANTILES = 1000
_NUM_TOKENS = 3
_N = 16384
_L = 200
_D = 64

_NC = 2
_NS = 16
_NW = _NC * _NS
_LANES = 16

_LT = _L // 8
_NT = _N // 128
_NTW = _NT // _NW
_BLK = _NTW * 8 * 128
_TROWS = 1024


def _embed_body(v5, i5, tabt_hbm, out5,
                vblk, iblk, outb, tab_v, lsem, osem0, osem1):
    cid = lax.axis_index("c")
    sid = lax.axis_index("s")
    wid = sid * _NC + cid
    coloff = wid * _NTW * 1024

    pltpu.sync_copy(tabt_hbm, tab_v)

    osem = (osem0, osem1)

    def fire_in(lt, boff):
        pltpu.async_copy(v5.at[lt, pl.ds(coloff, _BLK)],
                         vblk.at[pl.ds(boff, _BLK)], lsem)
        pltpu.async_copy(i5.at[lt, pl.ds(coloff, _BLK)],
                         iblk.at[pl.ds(boff, _BLK)], lsem)

    fire_in(0, 0)

    def lt_body(lt, carry):
        qoff = (lt % 2) * _BLK
        pltpu.make_async_copy(v5.at[lt, pl.ds(coloff, _BLK)],
                              vblk.at[pl.ds(qoff, _BLK)], lsem).wait()
        pltpu.make_async_copy(i5.at[lt, pl.ds(coloff, _BLK)],
                              iblk.at[pl.ds(qoff, _BLK)], lsem).wait()

        @pl.when(lt + 1 < _LT)
        def _():
            fire_in(lt + 1, _BLK - qoff)

        @plsc.parallel_loop(0, _BLK // _LANES, unroll=4)
        def _(m):
            v = vblk[pl.ds(qoff + m * _LANES, _LANES)]
            ind = iblk[pl.ds(qoff + m * _LANES, _LANES)]
            v = jnp.minimum(jnp.maximum(v, 0.0), 1.0)
            q = (v * float(_N_QUANTILES)).astype(jnp.int32) + _NUM_TOKENS
            lk = jnp.where(ind == 0, q, ind)
            lk = jnp.minimum(jnp.maximum(lk, 0),
                             _N_QUANTILES + _NUM_TOKENS - 1)
            iblk[pl.ds(qoff + m * _LANES, _LANES)] = lk

        def ls_body(ls, c3):
            l = lt * 8 + ls

            def dt3_body(dt3, c4):
                for e in range(2):
                    dt = 2 * dt3 + e
                    cnt = (lt * 8 + ls) * 8 + dt

                    @pl.when(cnt >= 2)
                    def _():
                        pltpu.make_async_copy(
                            outb.at[e],
                            out5.at[l, dt, pl.ds(wid * _NTW, _NTW)],
                            osem[e]).wait()

                    @plsc.parallel_loop(0, _NTW * 8, unroll=8)
                    def _(m):
                        nt4 = m // 8
                        g = m % 8
                        col = iblk[pl.ds(qoff + nt4 * 1024 + ls * 128
                                         + g * _LANES, _LANES)]
                        for ds in range(8):
                            x = plsc.load_gather(
                                tab_v, [col + (dt * 8 + ds) * _TROWS])
                            outb[e, nt4, ds, pl.ds(g * _LANES, _LANES)] = x
                    pltpu.async_copy(outb.at[e],
                                     out5.at[l, dt, pl.ds(wid * _NTW, _NTW)],
                                     osem[e])
                return c4

            lax.fori_loop(0, 4, dt3_body, 0)
            return c3

        lax.fori_loop(0, 8, ls_body, 0)
        return carry

    lax.fori_loop(0, _LT, lt_body, 0)

    for e in range(2):
        pltpu.make_async_copy(outb.at[e],
                              out5.at[_L - 1, 6 + e, pl.ds(wid * _NTW, _NTW)],
                              osem[e]).wait()


@jax.jit
def kernel(values, indicators, table):
    n, l = values.shape
    v5 = (values.reshape(_NT, 128, _LT, 8).transpose(2, 0, 3, 1)
          .reshape(_LT, _NT * 8 * 128))
    i5 = (indicators.reshape(_NT, 128, _LT, 8).transpose(2, 0, 3, 1)
          .reshape(_LT, _NT * 8 * 128))
    tabt = jnp.pad(table, ((0, _TROWS - table.shape[0]), (0, 0))).T.reshape(-1)

    run = functools.partial(
        pl.kernel,
        mesh=plsc.VectorSubcoreMesh(core_axis_name="c", subcore_axis_name="s"),
        compiler_params=pltpu.CompilerParams(use_tc_tiling_on_sc=False,
                                             needs_layout_passes=False),
        out_type=jax.ShapeDtypeStruct((_L, _D // 8, _NT, 8, 128),
                                      jnp.float32),
        scratch_types=[
            pltpu.VMEM((2 * _BLK,), jnp.float32),
            pltpu.VMEM((2 * _BLK,), jnp.int32),
            pltpu.VMEM((2, _NTW, 8, 128), jnp.float32),
            pltpu.VMEM((_D * _TROWS,), jnp.float32),
            pltpu.SemaphoreType.DMA,
            pltpu.SemaphoreType.DMA,
            pltpu.SemaphoreType.DMA,
        ],
    )(_embed_body)

    out5 = run(v5, i5, tabt)
    out = out5.transpose(2, 4, 0, 1, 3).reshape(n, l, _D)
    return out
